# baseline (speedup 1.0000x reference)
import functools

import jax
import jax.numpy as jnp
from jax import lax
from jax.experimental import pallas as pl
from jax.experimental.pallas import tpu as pltpu
from jax.experimental.pallas import tpu_sc as plsc

NC = 2
NS = 16
NW = NC * NS
LANES = 16

EMB = 64
E_SL = EMB // LANES
UBLK = 16
WBLK = 8
C = 16


def _flatten_seq(seq3d, *, B, L):
    BLK = 128

    def body(in_ref, out_ref):
        y = in_ref[...][:, :, 0]
        z = jnp.zeros((BLK, 128 - L), jnp.int32)
        out_ref[...] = jnp.concatenate([y, z], axis=1)

    return pl.pallas_call(
        body,
        grid=(B // BLK,),
        in_specs=[pl.BlockSpec((BLK, L, 1), lambda i: (i, 0, 0))],
        out_specs=pl.BlockSpec((BLK, 128), lambda i: (i, 0)),
        out_shape=jax.ShapeDtypeStruct((B, 128), jnp.int32),
    )(seq3d)


def _region_encode(seqp, W, U, *, B, L, R):
    TOK = B * L
    V = W.shape[0]
    b_per_w = B // NW
    per_w = b_per_w * L
    RAD = (R - 1) // 2
    NSL = C + 2 * RAD
    PADROW = NSL * UBLK
    n_chunks = per_w // C

    mesh = plsc.VectorSubcoreMesh(
        core_axis_name="c", subcore_axis_name="s", num_cores=NC, num_subcores=NS
    )

    @functools.partial(
        pl.kernel,
        out_type=jax.ShapeDtypeStruct((TOK, EMB), jnp.float32),
        mesh=mesh,
        compiler_params=pltpu.CompilerParams(needs_layout_passes=False),
        scratch_types=[
            pltpu.VMEM((b_per_w, 128), jnp.int32),
            pltpu.VMEM((2, (NSL + 1) * UBLK, EMB), jnp.float32),
            pltpu.VMEM((C, WBLK, EMB), jnp.float32),
            pltpu.VMEM((C, EMB), jnp.float32),
            pltpu.SemaphoreType.DMA,
            pltpu.SemaphoreType.DMA,
            pltpu.SemaphoreType.DMA,
        ],
    )
    def k(seq_hbm, W_hbm, U_hbm, out_hbm,
          seq_v, u_blk, w_blk, out_v, semw, semu0, semu1):
        wid = lax.axis_index("s") * NC + lax.axis_index("c")
        base = wid * per_w
        pltpu.sync_copy(seq_hbm.at[pl.ds(wid * b_per_w, b_per_w)], seq_v)
        pltpu.async_copy(
            U_hbm.at[pl.ds(0, UBLK)], u_blk.at[0, pl.ds(PADROW, UBLK)], semu0
        ).wait()
        pltpu.async_copy(
            U_hbm.at[pl.ds(0, UBLK)], u_blk.at[1, pl.ds(PADROW, UBLK)], semu1
        ).wait()

        semu = (semu0, semu1)
        lane = lax.broadcasted_iota(jnp.int32, (LANES,), 0)

        def slot_toks(c):
            q0 = c * C - RAD
            va_q = jnp.clip(q0 + lane, 0, per_w - 1)
            vb_q = jnp.clip(q0 + LANES + lane, 0, per_w - 1)
            va = plsc.load_gather(seq_v, [va_q // L, lax.rem(va_q, L)])
            vb = plsc.load_gather(seq_v, [vb_q // L, lax.rem(vb_q, L)])
            return va, vb

        def sv(va, vb, k_):
            return va[k_] if k_ < LANES else vb[k_ - LANES]

        def u_base(v):
            vr = v * R
            b8 = jnp.minimum(jnp.bitwise_and(vr, -8), V * R - UBLK)
            return pl.multiple_of(b8, 8)

        def w_base(v):
            b8 = jnp.minimum(jnp.bitwise_and(v, -8), V - WBLK)
            return pl.multiple_of(b8, 8)

        def fire_u(c, p):
            va, vb = slot_toks(c)
            for k_ in range(NSL):
                v = sv(va, vb, k_)
                pltpu.async_copy(
                    U_hbm.at[pl.ds(u_base(v), UBLK)],
                    u_blk.at[p, pl.ds(k_ * UBLK, UBLK)],
                    semu[p],
                )

        def drain_u(p):
            for k_ in range(NSL):
                pltpu.make_async_copy(
                    U_hbm.at[pl.ds(0, UBLK)],
                    u_blk.at[p, pl.ds(k_ * UBLK, UBLK)],
                    semu[p],
                ).wait()

        def fire_w(c):
            va, vb = slot_toks(c)
            for l in range(C):
                v = sv(va, vb, l + RAD)
                pltpu.async_copy(
                    W_hbm.at[pl.ds(w_base(v), WBLK)], w_blk.at[l], semw
                )

        def drain_w():
            for l in range(C):
                pltpu.make_async_copy(
                    W_hbm.at[pl.ds(0, WBLK)], w_blk.at[l], semw
                ).wait()

        def compute(c, p):
            va, vb = slot_toks(c)
            rowbase = []
            for k_ in range(NSL):
                v = sv(va, vb, k_)
                rowbase.append(k_ * UBLK + v * R - u_base(v))
            for l in range(C):
                v = sv(va, vb, l + RAD)
                lpos = lax.rem(c * C + l, L)
                wrow = v - w_base(v)
                maskf = jnp.where(v != 0, 1.0, 0.0).astype(jnp.float32)
                rows = []
                for i in range(R):
                    d = i - RAD
                    lq = lpos + d
                    valid = jnp.logical_and(lq >= 0, lq <= L - 1)
                    rows.append(
                        jnp.where(valid, rowbase[l + RAD + d], PADROW) + i
                    )
                for e in range(E_SL):
                    es = pl.ds(e * LANES, LANES)
                    w_e = w_blk[l, wrow, es]
                    acc = None
                    for i in range(R):
                        term = u_blk[p, rows[i], es] * w_e
                        acc = term if acc is None else jnp.maximum(acc, term)
                    out_v[l, es] = acc * maskf
            pltpu.sync_copy(out_v, out_hbm.at[pl.ds(base + c * C, C)])

        fire_u(0, 0)

        @pl.loop(0, n_chunks // 2)
        def pair_loop(t):
            c0 = 2 * t
            fire_u(c0 + 1, 1)
            fire_w(c0)
            drain_u(0)
            drain_w()
            compute(c0, 0)

            @pl.when(t < n_chunks // 2 - 1)
            def _():
                fire_u(c0 + 2, 0)

            fire_w(c0 + 1)
            drain_u(1)
            drain_w()
            compute(c0 + 1, 1)

    return k(seqp, W, U)


def kernel(seq, W, U):
    B, L, _ = seq.shape
    R = U.shape[0] // W.shape[0]
    seqp = _flatten_seq(seq, B=B, L=L)
    out = _region_encode(seqp, W, U, B=B, L=L, R=R)
    return out.reshape(B, L, 1, EMB)
